# Initial kernel scaffold; baseline (speedup 1.0000x reference)
import jax
import jax.numpy as jnp
from jax.experimental import pallas as pl


def kernel(x, edge_index, edge_attr, batch, W, att_src, att_dst, W_edge, att_edge, bias_conv, att_lin_W, att_lin_b, out_W, out_b):
    raise NotImplementedError("write your pallas kernel here")


# speedup vs baseline: 14.8301x; 14.8301x over previous
import functools

import jax
import jax.numpy as jnp
from jax import lax
from jax.experimental import pallas as pl
from jax.experimental.pallas import tpu as pltpu
from jax.experimental.pallas import tpu_sc as plsc

N = 100000
E = 1600000
F_IN = 12
H = 32
HH = 16
G = 128

NEG = -3.4e38
PAD_AE = -3.0e38

EPAD = 1638400
ER = EPAD // 128
ROWS_B = ER // 32
ROWS_D = ER // 16


def _a0_body(ea_ref, we_ref, ate_ref, ae_ref, csum_ref):
    i = pl.program_id(0)
    v3 = lax.dot_general(we_ref[...], ate_ref[...],
                         (((1,), (1,)), ((), ())))
    aeb = lax.dot_general(ea_ref[...], v3, (((1,), (0,)), ((), ())))
    ae_ref[...] = aeb

    @pl.when(i == 0)
    def _():
        csum_ref[...] = jnp.zeros_like(csum_ref)

    csum_ref[...] += jnp.sum(aeb, keepdims=True)


def _edge_attr_proj(edge_attr, W_edge, att_edge):
    BE = 8000
    grid = E // BE
    return pl.pallas_call(
        _a0_body,
        grid=(grid,),
        in_specs=[
            pl.BlockSpec((BE, 3), lambda i: (i, 0)),
            pl.BlockSpec((3, H), lambda i: (0, 0)),
            pl.BlockSpec((1, H), lambda i: (0, 0)),
        ],
        out_specs=[
            pl.BlockSpec((BE, 1), lambda i: (i, 0)),
            pl.BlockSpec((1, 1), lambda i: (0, 0)),
        ],
        out_shape=[
            jax.ShapeDtypeStruct((E, 1), jnp.float32),
            jax.ShapeDtypeStruct((1, 1), jnp.float32),
        ],
    )(edge_attr, W_edge, att_edge.reshape(1, H))


def _a1_body(x_ref, w_ref, asv_ref, adv_ref, c_ref,
             h0_ref, h1_ref, asrc_ref, adst_ref, sl_ref):
    h = jnp.dot(x_ref[...], w_ref[...], preferred_element_type=jnp.float32)
    h0_ref[...] = h[:, :HH]
    h1_ref[...] = h[:, HH:]
    a_s = jnp.sum(h * asv_ref[...], axis=1, keepdims=True)
    a_d = jnp.sum(h * adv_ref[...], axis=1, keepdims=True)
    asrc_ref[...] = a_s
    adst_ref[...] = a_d
    t = a_s + a_d + c_ref[...]
    sl_ref[...] = jnp.where(t >= 0, t, 0.2 * t)


def _node_proj(x, W, att_src, att_dst, c):
    BN = 2000
    grid = N // BN
    return pl.pallas_call(
        _a1_body,
        grid=(grid,),
        in_specs=[
            pl.BlockSpec((BN, F_IN), lambda i: (i, 0)),
            pl.BlockSpec((F_IN, H), lambda i: (0, 0)),
            pl.BlockSpec((1, H), lambda i: (0, 0)),
            pl.BlockSpec((1, H), lambda i: (0, 0)),
            pl.BlockSpec((1, 1), lambda i: (0, 0)),
        ],
        out_specs=[
            pl.BlockSpec((BN, HH), lambda i: (i, 0)),
            pl.BlockSpec((BN, HH), lambda i: (i, 0)),
            pl.BlockSpec((BN, 1), lambda i: (i, 0)),
            pl.BlockSpec((BN, 1), lambda i: (i, 0)),
            pl.BlockSpec((BN, 1), lambda i: (i, 0)),
        ],
        out_shape=[
            jax.ShapeDtypeStruct((N, HH), jnp.float32),
            jax.ShapeDtypeStruct((N, HH), jnp.float32),
            jax.ShapeDtypeStruct((N, 1), jnp.float32),
            jax.ShapeDtypeStruct((N, 1), jnp.float32),
            jax.ShapeDtypeStruct((N, 1), jnp.float32),
        ],
    )(x, W, att_src.reshape(1, H), att_dst.reshape(1, H), c)


def _c_body(mp_ref, sl_ref, h0_ref, h1_ref,
            m_ref, den0_ref, a00_ref, a01_ref):
    mp = jnp.max(mp_ref[...], axis=1)
    sl = sl_ref[...][:, 0]
    m = jnp.maximum(mp, sl)
    selfex = jnp.exp(sl - m)
    m_ref[...] = m[:, None]
    den0_ref[...] = selfex[:, None]
    a00_ref[...] = h0_ref[...] * selfex[:, None]
    a01_ref[...] = h1_ref[...] * selfex[:, None]


def _combine_max(mpart, sl, h0, h1):
    BN = 1000
    grid = N // BN
    return pl.pallas_call(
        _c_body,
        grid=(grid,),
        in_specs=[
            pl.BlockSpec((BN, 32), lambda i: (i, 0)),
            pl.BlockSpec((BN, 1), lambda i: (i, 0)),
            pl.BlockSpec((BN, HH), lambda i: (i, 0)),
            pl.BlockSpec((BN, HH), lambda i: (i, 0)),
        ],
        out_specs=[
            pl.BlockSpec((BN, 1), lambda i: (i, 0)),
            pl.BlockSpec((BN, 1), lambda i: (i, 0)),
            pl.BlockSpec((BN, HH), lambda i: (i, 0)),
            pl.BlockSpec((BN, HH), lambda i: (i, 0)),
        ],
        out_shape=[
            jax.ShapeDtypeStruct((N, 1), jnp.float32),
            jax.ShapeDtypeStruct((N, 1), jnp.float32),
            jax.ShapeDtypeStruct((N, HH), jnp.float32),
            jax.ShapeDtypeStruct((N, HH), jnp.float32),
        ],
    )(mpart, sl, h0, h1)


def _e1_body(n0_ref, n1_ref, den_ref, bias_ref, alw_ref, alb_ref, b_ref,
             out_ref, gate_ref, mg_ref):
    i = pl.program_id(0)
    acc = jnp.concatenate([n0_ref[...], n1_ref[...]], axis=1)
    outv = acc / (den_ref[...] + 1e-16) + bias_ref[...]
    outv = jnp.maximum(outv, 0.0)
    out_ref[...] = outv
    gate = jnp.sum(outv * alw_ref[...], axis=1, keepdims=True) + alb_ref[...]
    gate_ref[...] = gate

    @pl.when(i == 0)
    def _():
        mg_ref[...] = jnp.full_like(mg_ref, NEG)

    iota = lax.broadcasted_iota(jnp.int32, (gate.shape[0], G), 1)
    mask = iota == b_ref[...]
    gv = jnp.where(mask, gate, NEG)
    mg_ref[...] = jnp.maximum(mg_ref[...], jnp.max(gv, axis=0, keepdims=True))


def _node_out(num0, num1, den, bias, alw, alb, batch):
    BN = 1000
    grid = N // BN
    return pl.pallas_call(
        _e1_body,
        grid=(grid,),
        in_specs=[
            pl.BlockSpec((BN, HH), lambda i: (i, 0)),
            pl.BlockSpec((BN, HH), lambda i: (i, 0)),
            pl.BlockSpec((BN, 1), lambda i: (i, 0)),
            pl.BlockSpec((1, H), lambda i: (0, 0)),
            pl.BlockSpec((1, H), lambda i: (0, 0)),
            pl.BlockSpec((1, 1), lambda i: (0, 0)),
            pl.BlockSpec((BN, 1), lambda i: (i, 0)),
        ],
        out_specs=[
            pl.BlockSpec((BN, H), lambda i: (i, 0)),
            pl.BlockSpec((BN, 1), lambda i: (i, 0)),
            pl.BlockSpec((1, G), lambda i: (0, 0)),
        ],
        out_shape=[
            jax.ShapeDtypeStruct((N, H), jnp.float32),
            jax.ShapeDtypeStruct((N, 1), jnp.float32),
            jax.ShapeDtypeStruct((1, G), jnp.float32),
        ],
    )(num0, num1, den, bias, alw, alb, batch)


def _e2_body(out_ref, gate_ref, b_ref, mg_ref, ow_ref, ob_ref,
             pn_ref, pd_ref, y_ref):
    i = pl.program_id(0)
    ng = pl.num_programs(0)

    @pl.when(i == 0)
    def _():
        pn_ref[...] = jnp.zeros_like(pn_ref)
        pd_ref[...] = jnp.zeros_like(pd_ref)

    mg = mg_ref[...]
    mgs = jnp.where(mg > -1e38, mg, 0.0)
    iota = lax.broadcasted_iota(jnp.int32, (out_ref.shape[0], G), 1)
    bo = (iota == b_ref[...]).astype(jnp.float32)
    mgb = jnp.sum(bo * mgs, axis=1, keepdims=True)
    w = jnp.exp(gate_ref[...] - mgb)
    oww = out_ref[...] * w
    pn_ref[...] += lax.dot_general(bo, oww, (((0,), (0,)), ((), ())),
                                   preferred_element_type=jnp.float32)
    pd_ref[...] += lax.dot_general(bo, w, (((0,), (0,)), ((), ())),
                                   preferred_element_type=jnp.float32)

    @pl.when(i == ng - 1)
    def _():
        pool = pn_ref[...] / (pd_ref[...] + 1e-16)
        yv = jnp.dot(pool, ow_ref[...], preferred_element_type=jnp.float32)
        yv = yv + ob_ref[...]
        y_ref[...] = 1.0 / (1.0 + jnp.exp(-yv))


def _pool(out, gate, batch, mg, out_W, out_b):
    BN = 1000
    grid = N // BN
    return pl.pallas_call(
        _e2_body,
        grid=(grid,),
        in_specs=[
            pl.BlockSpec((BN, H), lambda i: (i, 0)),
            pl.BlockSpec((BN, 1), lambda i: (i, 0)),
            pl.BlockSpec((BN, 1), lambda i: (i, 0)),
            pl.BlockSpec((1, G), lambda i: (0, 0)),
            pl.BlockSpec((H, 1), lambda i: (0, 0)),
            pl.BlockSpec((1, 1), lambda i: (0, 0)),
        ],
        out_specs=[
            pl.BlockSpec((G, H), lambda i: (0, 0)),
            pl.BlockSpec((G, 1), lambda i: (0, 0)),
            pl.BlockSpec((G, 1), lambda i: (0, 0)),
        ],
        out_shape=[
            jax.ShapeDtypeStruct((G, H), jnp.float32),
            jax.ShapeDtypeStruct((G, 1), jnp.float32),
            jax.ShapeDtypeStruct((G, 1), jnp.float32),
        ],
    )(out, gate, batch, mg, out_W, out_b)


_MESH = plsc.VectorSubcoreMesh(core_axis_name="c", subcore_axis_name="s")
_SC_PARAMS = pltpu.CompilerParams(needs_layout_passes=False,
                                  use_tc_tiling_on_sc=False)


def _sc_pass1(src_hbm, dst_hbm, ae_hbm, asrc_hbm, adst_hbm,
              l_hbm, mpart_hbm,
              asrc_sh, adst_sh, m_t, srcb, dstb, aeb, lb, av, dv, shb):
    c = lax.axis_index("c")
    s = lax.axis_index("s")
    wid = c * 16 + s

    @pl.when(s == 0)
    def _():
        pltpu.sync_copy(asrc_hbm, asrc_sh)
        pltpu.sync_copy(adst_hbm, adst_sh)

    @pl.loop(0, N // 16)
    def _(i):
        m_t[pl.ds(i * 16, 16)] = jnp.full((16,), NEG, jnp.float32)

    shb[pl.ds(0, 16)] = jnp.full((16,), -1.0, jnp.float32)
    shb[pl.ds(32, 16)] = jnp.full((16,), -2.0, jnp.float32)

    plsc.subcore_barrier()

    rowbase = wid * ROWS_B

    @pl.loop(0, ROWS_B // 8)
    def _(ch):
        rb = rowbase + ch * 8
        pltpu.sync_copy(src_hbm.at[pl.ds(rb, 8)], srcb)
        pltpu.sync_copy(dst_hbm.at[pl.ds(rb, 8)], dstb)
        pltpu.sync_copy(ae_hbm.at[pl.ds(rb, 8)], aeb)

        @pl.loop(0, 8)
        def _(k):
            pltpu.sync_copy(asrc_sh.at[srcb.at[k]], av)
            pltpu.sync_copy(adst_sh.at[dstb.at[k]], dv)

            @pl.loop(0, 8)
            def _(j):
                sl16 = pl.ds(j * 16, 16)
                a = av[sl16] + dv[sl16] + aeb[k, sl16]
                lb[k, sl16] = jnp.where(a >= 0, a, 0.2 * a)

            @pl.loop(0, 8)
            def _(j):
                sl16 = pl.ds(j * 16, 16)
                kk, vv = plsc.sort_key_val(dstb[k, sl16], lb[k, sl16])
                kf = kk.astype(jnp.float32)
                shb[pl.ds(16, 16)] = kf
                shb[pl.ds(48, 16)] = vv
                for sh in (1, 2, 4, 8):
                    pk = shb[pl.ds(16 - sh, 16)]
                    pv = shb[pl.ds(48 - sh, 16)]
                    take = pk == kf
                    vv = jnp.where(take, jnp.maximum(vv, pv), vv)
                    if sh < 8:
                        shb[pl.ds(48, 16)] = vv
                nk = shb[pl.ds(17, 16)]
                islast = nk != kf
                cur = plsc.load_gather(m_t, [kk])
                plsc.store_scatter(m_t, [kk], jnp.maximum(cur, vv),
                                   mask=islast)

        pltpu.sync_copy(lb, l_hbm.at[pl.ds(rb, 8)])

    pltpu.sync_copy(m_t, mpart_hbm.at[wid])


def _run_pass1(srcp, dstp, aep, asrc1, adst1):
    kfn = pl.kernel(
        _sc_pass1,
        out_type=[
            jax.ShapeDtypeStruct((ER, 128), jnp.float32),
            jax.ShapeDtypeStruct((32, N), jnp.float32),
        ],
        mesh=_MESH,
        scratch_types=[
            pltpu.MemorySpace.VMEM_SHARED((N,), jnp.float32),
            pltpu.MemorySpace.VMEM_SHARED((N,), jnp.float32),
            pltpu.MemorySpace.VMEM((N,), jnp.float32),
            pltpu.MemorySpace.VMEM((8, 128), jnp.int32),
            pltpu.MemorySpace.VMEM((8, 128), jnp.int32),
            pltpu.MemorySpace.VMEM((8, 128), jnp.float32),
            pltpu.MemorySpace.VMEM((8, 128), jnp.float32),
            pltpu.MemorySpace.VMEM((128,), jnp.float32),
            pltpu.MemorySpace.VMEM((128,), jnp.float32),
            pltpu.MemorySpace.VMEM((64,), jnp.float32),
        ],
        compiler_params=_SC_PARAMS,
    )
    return kfn(srcp, dstp, aep, asrc1, adst1)


def _sc_pass2(src_hbm, dst_hbm, l_hbm, m_hbm, h01_hbm, acc0_hbm, den0_hbm,
              num_hbm, den_hbm,
              acc_sh, den_sh, m_sh, srcb, dstb, lbuf, mv, exv, rbuf):
    c = lax.axis_index("c")
    s = lax.axis_index("s")
    cN = c * N
    rows0 = s * (N // 16)

    pltpu.sync_copy(acc0_hbm.at[pl.ds(cN + rows0, N // 16)],
                    acc_sh.at[pl.ds(rows0, N // 16)])

    @pl.when(s == 0)
    def _():
        pltpu.sync_copy(m_hbm, m_sh)

    @pl.when(s == 1)
    def _():
        pltpu.sync_copy(den0_hbm, den_sh)

    plsc.subcore_barrier()

    rowbase = s * ROWS_D

    @pl.loop(0, ROWS_D // 8)
    def _(ch):
        rb = rowbase + ch * 8
        pltpu.sync_copy(src_hbm.at[pl.ds(rb, 8)], srcb)
        pltpu.sync_copy(dst_hbm.at[pl.ds(rb, 8)], dstb)
        pltpu.sync_copy(l_hbm.at[pl.ds(rb, 8)], lbuf)

        @pl.loop(0, 8)
        def _(k):
            pltpu.sync_copy(m_sh.at[dstb.at[k]], mv)

            @pl.loop(0, 8)
            def _(j):
                sl16 = pl.ds(j * 16, 16)
                srcb[k, sl16] = srcb[k, sl16] + cN
                exv[sl16] = jnp.exp(lbuf[k, sl16] - mv[sl16])

            pltpu.sync_copy(h01_hbm.at[srcb.at[k]], rbuf)

            @pl.loop(0, 8)
            def _(q):
                exq = exv[pl.ds(q * 16, 16)]
                for t in range(16):
                    jj = q * 16 + t
                    rbuf[jj, pl.ds(0, HH)] = rbuf[jj, pl.ds(0, HH)] * exq[t]

            pltpu.sync_copy(rbuf, acc_sh.at[dstb.at[k]], add=True)

            @pl.when(c == 0)
            def _():
                pltpu.sync_copy(exv, den_sh.at[dstb.at[k]], add=True)

    plsc.subcore_barrier()

    pltpu.sync_copy(acc_sh.at[pl.ds(rows0, N // 16)],
                    num_hbm.at[pl.ds(cN + rows0, N // 16)])

    @pl.when(jnp.logical_and(c == 0, s == 0))
    def _():
        pltpu.sync_copy(den_sh, den_hbm)


def _run_pass2(srcp, dstp, l2d, m1, h01, acc0, den01):
    kfn = pl.kernel(
        _sc_pass2,
        out_type=[
            jax.ShapeDtypeStruct((2 * N, HH), jnp.float32),
            jax.ShapeDtypeStruct((N,), jnp.float32),
        ],
        mesh=_MESH,
        scratch_types=[
            pltpu.MemorySpace.VMEM_SHARED((N, HH), jnp.float32),
            pltpu.MemorySpace.VMEM_SHARED((N,), jnp.float32),
            pltpu.MemorySpace.VMEM_SHARED((N,), jnp.float32),
            pltpu.MemorySpace.VMEM((8, 128), jnp.int32),
            pltpu.MemorySpace.VMEM((8, 128), jnp.int32),
            pltpu.MemorySpace.VMEM((8, 128), jnp.float32),
            pltpu.MemorySpace.VMEM((128,), jnp.float32),
            pltpu.MemorySpace.VMEM((128,), jnp.float32),
            pltpu.MemorySpace.VMEM((128, HH), jnp.float32),
        ],
        compiler_params=_SC_PARAMS,
    )
    return kfn(srcp, dstp, l2d, m1, h01, acc0, den01)


@jax.jit
def kernel(x, edge_index, edge_attr, batch, W, att_src, att_dst, W_edge,
           att_edge, bias_conv, att_lin_W, att_lin_b, out_W, out_b):
    ae, csum = _edge_attr_proj(edge_attr, W_edge, att_edge)
    c = csum / jnp.float32(E)

    h0, h1, asrc, adst, sl = _node_proj(x, W, att_src, att_dst, c)

    src = edge_index[0]
    dst = edge_index[1]
    pad = EPAD - E
    srcp = jnp.pad(src, (0, pad)).reshape(ER, 128)
    dstp = jnp.pad(dst, (0, pad)).reshape(ER, 128)
    aep = jnp.pad(ae[:, 0], (0, pad), constant_values=PAD_AE).reshape(ER, 128)

    l2d, mpart = _run_pass1(srcp, dstp, aep,
                            asrc.reshape(N), adst.reshape(N))

    m, den0, a00, a01 = _combine_max(jnp.transpose(mpart), sl, h0, h1)

    h01 = jnp.concatenate([h0, h1], axis=0)
    acc0 = jnp.concatenate([a00, a01], axis=0)

    num, den = _run_pass2(srcp, dstp, l2d, m.reshape(N), h01, acc0,
                          den0.reshape(N))

    out, gate, mg = _node_out(num[:N], num[N:], den.reshape(N, 1),
                              bias_conv.reshape(1, H),
                              att_lin_W.reshape(1, H),
                              att_lin_b.reshape(1, 1),
                              batch.reshape(N, 1))

    _, _, y = _pool(out, gate, batch.reshape(N, 1), mg,
                    out_W, out_b.reshape(1, 1))
    return y


# speedup vs baseline: 17.6695x; 1.1915x over previous
import functools

import jax
import jax.numpy as jnp
from jax import lax
from jax.experimental import pallas as pl
from jax.experimental.pallas import tpu as pltpu
from jax.experimental.pallas import tpu_sc as plsc

N = 100000
E = 1600000
F_IN = 12
H = 32
HH = 16
G = 128

NEG = -3.4e38
PAD_AE = -3.0e38

EPAD = 1638400
ER = EPAD // 128
ROWS_B = ER // 32
ROWS_D = ER // 16


def _a0_body(ea_ref, we_ref, ate_ref, ae_ref, csum_ref):
    i = pl.program_id(0)
    v3 = lax.dot_general(we_ref[...], ate_ref[...],
                         (((1,), (1,)), ((), ())))
    aeb = lax.dot_general(ea_ref[...], v3, (((1,), (0,)), ((), ())))
    ae_ref[...] = aeb

    @pl.when(i == 0)
    def _():
        csum_ref[...] = jnp.zeros_like(csum_ref)

    csum_ref[...] += jnp.sum(aeb, keepdims=True)


def _edge_attr_proj(edge_attr, W_edge, att_edge):
    BE = 8000
    grid = E // BE
    return pl.pallas_call(
        _a0_body,
        grid=(grid,),
        in_specs=[
            pl.BlockSpec((BE, 3), lambda i: (i, 0)),
            pl.BlockSpec((3, H), lambda i: (0, 0)),
            pl.BlockSpec((1, H), lambda i: (0, 0)),
        ],
        out_specs=[
            pl.BlockSpec((BE, 1), lambda i: (i, 0)),
            pl.BlockSpec((1, 1), lambda i: (0, 0)),
        ],
        out_shape=[
            jax.ShapeDtypeStruct((E, 1), jnp.float32),
            jax.ShapeDtypeStruct((1, 1), jnp.float32),
        ],
    )(edge_attr, W_edge, att_edge.reshape(1, H))


def _a1_body(x_ref, w_ref, asv_ref, adv_ref, c_ref,
             h0_ref, h1_ref, asrc_ref, adst_ref, sl_ref):
    h = jnp.dot(x_ref[...], w_ref[...], preferred_element_type=jnp.float32)
    h0_ref[...] = h[:, :HH]
    h1_ref[...] = h[:, HH:]
    a_s = jnp.sum(h * asv_ref[...], axis=1, keepdims=True)
    a_d = jnp.sum(h * adv_ref[...], axis=1, keepdims=True)
    asrc_ref[...] = a_s
    adst_ref[...] = a_d
    t = a_s + a_d + c_ref[...]
    sl_ref[...] = jnp.where(t >= 0, t, 0.2 * t)


def _node_proj(x, W, att_src, att_dst, c):
    BN = 2000
    grid = N // BN
    return pl.pallas_call(
        _a1_body,
        grid=(grid,),
        in_specs=[
            pl.BlockSpec((BN, F_IN), lambda i: (i, 0)),
            pl.BlockSpec((F_IN, H), lambda i: (0, 0)),
            pl.BlockSpec((1, H), lambda i: (0, 0)),
            pl.BlockSpec((1, H), lambda i: (0, 0)),
            pl.BlockSpec((1, 1), lambda i: (0, 0)),
        ],
        out_specs=[
            pl.BlockSpec((BN, HH), lambda i: (i, 0)),
            pl.BlockSpec((BN, HH), lambda i: (i, 0)),
            pl.BlockSpec((BN, 1), lambda i: (i, 0)),
            pl.BlockSpec((BN, 1), lambda i: (i, 0)),
            pl.BlockSpec((BN, 1), lambda i: (i, 0)),
        ],
        out_shape=[
            jax.ShapeDtypeStruct((N, HH), jnp.float32),
            jax.ShapeDtypeStruct((N, HH), jnp.float32),
            jax.ShapeDtypeStruct((N, 1), jnp.float32),
            jax.ShapeDtypeStruct((N, 1), jnp.float32),
            jax.ShapeDtypeStruct((N, 1), jnp.float32),
        ],
    )(x, W, att_src.reshape(1, H), att_dst.reshape(1, H), c)


def _c_body(mp_ref, sl_ref, h0_ref, h1_ref,
            m_ref, den0_ref, a00_ref, a01_ref):
    mp = jnp.max(mp_ref[...], axis=1)
    sl = sl_ref[...][:, 0]
    m = jnp.maximum(mp, sl)
    selfex = jnp.exp(sl - m)
    m_ref[...] = m[:, None]
    den0_ref[...] = selfex[:, None]
    a00_ref[...] = h0_ref[...] * selfex[:, None]
    a01_ref[...] = h1_ref[...] * selfex[:, None]


def _combine_max(mpart, sl, h0, h1):
    BN = 1000
    grid = N // BN
    return pl.pallas_call(
        _c_body,
        grid=(grid,),
        in_specs=[
            pl.BlockSpec((BN, 32), lambda i: (i, 0)),
            pl.BlockSpec((BN, 1), lambda i: (i, 0)),
            pl.BlockSpec((BN, HH), lambda i: (i, 0)),
            pl.BlockSpec((BN, HH), lambda i: (i, 0)),
        ],
        out_specs=[
            pl.BlockSpec((BN, 1), lambda i: (i, 0)),
            pl.BlockSpec((BN, 1), lambda i: (i, 0)),
            pl.BlockSpec((BN, HH), lambda i: (i, 0)),
            pl.BlockSpec((BN, HH), lambda i: (i, 0)),
        ],
        out_shape=[
            jax.ShapeDtypeStruct((N, 1), jnp.float32),
            jax.ShapeDtypeStruct((N, 1), jnp.float32),
            jax.ShapeDtypeStruct((N, HH), jnp.float32),
            jax.ShapeDtypeStruct((N, HH), jnp.float32),
        ],
    )(mpart, sl, h0, h1)


def _e1_body(n0_ref, n1_ref, den_ref, bias_ref, alw_ref, alb_ref, b_ref,
             out_ref, gate_ref, mg_ref):
    i = pl.program_id(0)
    acc = jnp.concatenate([n0_ref[...], n1_ref[...]], axis=1)
    outv = acc / (den_ref[...] + 1e-16) + bias_ref[...]
    outv = jnp.maximum(outv, 0.0)
    out_ref[...] = outv
    gate = jnp.sum(outv * alw_ref[...], axis=1, keepdims=True) + alb_ref[...]
    gate_ref[...] = gate

    @pl.when(i == 0)
    def _():
        mg_ref[...] = jnp.full_like(mg_ref, NEG)

    iota = lax.broadcasted_iota(jnp.int32, (gate.shape[0], G), 1)
    mask = iota == b_ref[...]
    gv = jnp.where(mask, gate, NEG)
    mg_ref[...] = jnp.maximum(mg_ref[...], jnp.max(gv, axis=0, keepdims=True))


def _node_out(num0, num1, den, bias, alw, alb, batch):
    BN = 1000
    grid = N // BN
    return pl.pallas_call(
        _e1_body,
        grid=(grid,),
        in_specs=[
            pl.BlockSpec((BN, HH), lambda i: (i, 0)),
            pl.BlockSpec((BN, HH), lambda i: (i, 0)),
            pl.BlockSpec((BN, 1), lambda i: (i, 0)),
            pl.BlockSpec((1, H), lambda i: (0, 0)),
            pl.BlockSpec((1, H), lambda i: (0, 0)),
            pl.BlockSpec((1, 1), lambda i: (0, 0)),
            pl.BlockSpec((BN, 1), lambda i: (i, 0)),
        ],
        out_specs=[
            pl.BlockSpec((BN, H), lambda i: (i, 0)),
            pl.BlockSpec((BN, 1), lambda i: (i, 0)),
            pl.BlockSpec((1, G), lambda i: (0, 0)),
        ],
        out_shape=[
            jax.ShapeDtypeStruct((N, H), jnp.float32),
            jax.ShapeDtypeStruct((N, 1), jnp.float32),
            jax.ShapeDtypeStruct((1, G), jnp.float32),
        ],
    )(num0, num1, den, bias, alw, alb, batch)


def _e2_body(out_ref, gate_ref, b_ref, mg_ref, ow_ref, ob_ref,
             pn_ref, pd_ref, y_ref):
    i = pl.program_id(0)
    ng = pl.num_programs(0)

    @pl.when(i == 0)
    def _():
        pn_ref[...] = jnp.zeros_like(pn_ref)
        pd_ref[...] = jnp.zeros_like(pd_ref)

    mg = mg_ref[...]
    mgs = jnp.where(mg > -1e38, mg, 0.0)
    iota = lax.broadcasted_iota(jnp.int32, (out_ref.shape[0], G), 1)
    bo = (iota == b_ref[...]).astype(jnp.float32)
    mgb = jnp.sum(bo * mgs, axis=1, keepdims=True)
    w = jnp.exp(gate_ref[...] - mgb)
    oww = out_ref[...] * w
    pn_ref[...] += lax.dot_general(bo, oww, (((0,), (0,)), ((), ())),
                                   preferred_element_type=jnp.float32)
    pd_ref[...] += lax.dot_general(bo, w, (((0,), (0,)), ((), ())),
                                   preferred_element_type=jnp.float32)

    @pl.when(i == ng - 1)
    def _():
        pool = pn_ref[...] / (pd_ref[...] + 1e-16)
        yv = jnp.dot(pool, ow_ref[...], preferred_element_type=jnp.float32)
        yv = yv + ob_ref[...]
        y_ref[...] = 1.0 / (1.0 + jnp.exp(-yv))


def _pool(out, gate, batch, mg, out_W, out_b):
    BN = 1000
    grid = N // BN
    return pl.pallas_call(
        _e2_body,
        grid=(grid,),
        in_specs=[
            pl.BlockSpec((BN, H), lambda i: (i, 0)),
            pl.BlockSpec((BN, 1), lambda i: (i, 0)),
            pl.BlockSpec((BN, 1), lambda i: (i, 0)),
            pl.BlockSpec((1, G), lambda i: (0, 0)),
            pl.BlockSpec((H, 1), lambda i: (0, 0)),
            pl.BlockSpec((1, 1), lambda i: (0, 0)),
        ],
        out_specs=[
            pl.BlockSpec((G, H), lambda i: (0, 0)),
            pl.BlockSpec((G, 1), lambda i: (0, 0)),
            pl.BlockSpec((G, 1), lambda i: (0, 0)),
        ],
        out_shape=[
            jax.ShapeDtypeStruct((G, H), jnp.float32),
            jax.ShapeDtypeStruct((G, 1), jnp.float32),
            jax.ShapeDtypeStruct((G, 1), jnp.float32),
        ],
    )(out, gate, batch, mg, out_W, out_b)


_MESH = plsc.VectorSubcoreMesh(core_axis_name="c", subcore_axis_name="s")
_SC_PARAMS = pltpu.CompilerParams(needs_layout_passes=False,
                                  use_tc_tiling_on_sc=False)


def _sc_pass1(src_hbm, dst_hbm, ae_hbm, asrc_hbm, adst_hbm,
              l_hbm, mpart_hbm,
              asrc_sh, adst_sh, m_t, srcb, dstb, aeb, lb, av, dv, shb):
    c = lax.axis_index("c")
    s = lax.axis_index("s")
    wid = c * 16 + s

    @pl.when(s == 0)
    def _():
        pltpu.sync_copy(asrc_hbm, asrc_sh)
        pltpu.sync_copy(adst_hbm, adst_sh)

    @pl.loop(0, N // 16)
    def _(i):
        m_t[pl.ds(i * 16, 16)] = jnp.full((16,), NEG, jnp.float32)

    shb[pl.ds(0, 16)] = jnp.full((16,), -1.0, jnp.float32)
    shb[pl.ds(32, 16)] = jnp.full((16,), -2.0, jnp.float32)

    plsc.subcore_barrier()

    rowbase = wid * ROWS_B

    @pl.loop(0, ROWS_B // 8)
    def _(ch):
        rb = rowbase + ch * 8
        pltpu.sync_copy(src_hbm.at[pl.ds(rb, 8)], srcb)
        pltpu.sync_copy(dst_hbm.at[pl.ds(rb, 8)], dstb)
        pltpu.sync_copy(ae_hbm.at[pl.ds(rb, 8)], aeb)

        @pl.loop(0, 8)
        def _(k):
            pltpu.sync_copy(asrc_sh.at[srcb.at[k]], av)
            pltpu.sync_copy(adst_sh.at[dstb.at[k]], dv)

            @pl.loop(0, 8)
            def _(j):
                sl16 = pl.ds(j * 16, 16)
                a = av[sl16] + dv[sl16] + aeb[k, sl16]
                lb[k, sl16] = jnp.where(a >= 0, a, 0.2 * a)

            @pl.loop(0, 8)
            def _(j):
                sl16 = pl.ds(j * 16, 16)
                kk, vv = plsc.sort_key_val(dstb[k, sl16], lb[k, sl16])
                kf = kk.astype(jnp.float32)
                shb[pl.ds(16, 16)] = kf
                shb[pl.ds(48, 16)] = vv
                for sh in (1, 2, 4, 8):
                    pk = shb[pl.ds(16 - sh, 16)]
                    pv = shb[pl.ds(48 - sh, 16)]
                    take = pk == kf
                    vv = jnp.where(take, jnp.maximum(vv, pv), vv)
                    if sh < 8:
                        shb[pl.ds(48, 16)] = vv
                nk = shb[pl.ds(17, 16)]
                islast = nk != kf
                cur = plsc.load_gather(m_t, [kk])
                plsc.store_scatter(m_t, [kk], jnp.maximum(cur, vv),
                                   mask=islast)

        pltpu.sync_copy(lb, l_hbm.at[pl.ds(rb, 8)])

    pltpu.sync_copy(m_t, mpart_hbm.at[wid])


def _run_pass1(srcp, dstp, aep, asrc1, adst1):
    kfn = pl.kernel(
        _sc_pass1,
        out_type=[
            jax.ShapeDtypeStruct((ER, 128), jnp.float32),
            jax.ShapeDtypeStruct((32, N), jnp.float32),
        ],
        mesh=_MESH,
        scratch_types=[
            pltpu.MemorySpace.VMEM_SHARED((N,), jnp.float32),
            pltpu.MemorySpace.VMEM_SHARED((N,), jnp.float32),
            pltpu.MemorySpace.VMEM((N,), jnp.float32),
            pltpu.MemorySpace.VMEM((8, 128), jnp.int32),
            pltpu.MemorySpace.VMEM((8, 128), jnp.int32),
            pltpu.MemorySpace.VMEM((8, 128), jnp.float32),
            pltpu.MemorySpace.VMEM((8, 128), jnp.float32),
            pltpu.MemorySpace.VMEM((128,), jnp.float32),
            pltpu.MemorySpace.VMEM((128,), jnp.float32),
            pltpu.MemorySpace.VMEM((64,), jnp.float32),
        ],
        compiler_params=_SC_PARAMS,
    )
    return kfn(srcp, dstp, aep, asrc1, adst1)


def _sc_pass2(src_hbm, dst_hbm, l_hbm, m_hbm, h01_hbm, acc0_hbm, den0_hbm,
              num_hbm, den_hbm,
              acc_sh, den_sh, m_sh, srcb, dstb, lbuf, mvb, exb, rbig,
              gs0, gs1, gs2, vsem, as0, as1, as2):
    gsems = (gs0, gs1, gs2)
    asems = (as0, as1, as2)
    c = lax.axis_index("c")
    s = lax.axis_index("s")
    cN = c * N
    rows0 = s * (N // 16)

    pltpu.sync_copy(acc0_hbm.at[pl.ds(cN + rows0, N // 16)],
                    acc_sh.at[pl.ds(rows0, N // 16)])

    @pl.when(s == 0)
    def _():
        pltpu.sync_copy(m_hbm, m_sh)

    @pl.when(s == 1)
    def _():
        pltpu.sync_copy(den0_hbm, den_sh)

    plsc.subcore_barrier()

    rowbase = s * ROWS_D

    @pl.loop(0, ROWS_D // 8)
    def _(ch):
        rb = rowbase + ch * 8
        pltpu.sync_copy(src_hbm.at[pl.ds(rb, 8)], srcb)
        pltpu.sync_copy(dst_hbm.at[pl.ds(rb, 8)], dstb)
        pltpu.sync_copy(l_hbm.at[pl.ds(rb, 8)], lbuf)

        @pl.loop(0, 8)
        def _(k):
            @pl.loop(0, 8)
            def _(j):
                sl16 = pl.ds(j * 16, 16)
                srcb[k, sl16] = srcb[k, sl16] + cN

        mv_d = [pltpu.async_copy(m_sh.at[dstb.at[k]], mvb.at[k], vsem)
                for k in range(8)]
        for d in mv_d:
            d.wait()
        g_d = [None] * 8
        for k in range(2):
            g_d[k] = pltpu.async_copy(h01_hbm.at[srcb.at[k]],
                                      rbig.at[k % 3], gsems[k % 3])
        sc_d = [None] * 8
        for k in range(8):
            @pl.loop(0, 8)
            def _(j, k=k):
                sl16 = pl.ds(j * 16, 16)
                exb[k, sl16] = jnp.exp(lbuf[k, sl16] - mvb[k, sl16])

            g_d[k].wait()
            if k + 2 < 8:
                if k >= 1:
                    sc_d[k - 1].wait()
                g_d[k + 2] = pltpu.async_copy(h01_hbm.at[srcb.at[k + 2]],
                                              rbig.at[(k + 2) % 3],
                                              gsems[(k + 2) % 3])

            @pl.loop(0, 8)
            def _(q, k=k):
                exq = exb[k, pl.ds(q * 16, 16)]
                for t in range(16):
                    jj = q * 16 + t
                    rbig[k % 3, jj, pl.ds(0, HH)] = (
                        rbig[k % 3, jj, pl.ds(0, HH)] * exq[t])

            sc_d[k] = pltpu.async_copy(rbig.at[k % 3], acc_sh.at[dstb.at[k]],
                                       asems[k % 3], add=True)

            @pl.when(c == 0)
            def _(k=k):
                pltpu.sync_copy(exb.at[k], den_sh.at[dstb.at[k]], add=True)

        sc_d[5].wait()
        sc_d[6].wait()
        sc_d[7].wait()

    plsc.subcore_barrier()

    pltpu.sync_copy(acc_sh.at[pl.ds(rows0, N // 16)],
                    num_hbm.at[pl.ds(cN + rows0, N // 16)])

    @pl.when(jnp.logical_and(c == 0, s == 0))
    def _():
        pltpu.sync_copy(den_sh, den_hbm)


def _run_pass2(srcp, dstp, l2d, m1, h01, acc0, den01):
    kfn = pl.kernel(
        _sc_pass2,
        out_type=[
            jax.ShapeDtypeStruct((2 * N, HH), jnp.float32),
            jax.ShapeDtypeStruct((N,), jnp.float32),
        ],
        mesh=_MESH,
        scratch_types=[
            pltpu.MemorySpace.VMEM_SHARED((N, HH), jnp.float32),
            pltpu.MemorySpace.VMEM_SHARED((N,), jnp.float32),
            pltpu.MemorySpace.VMEM_SHARED((N,), jnp.float32),
            pltpu.MemorySpace.VMEM((8, 128), jnp.int32),
            pltpu.MemorySpace.VMEM((8, 128), jnp.int32),
            pltpu.MemorySpace.VMEM((8, 128), jnp.float32),
            pltpu.MemorySpace.VMEM((8, 128), jnp.float32),
            pltpu.MemorySpace.VMEM((8, 128), jnp.float32),
            pltpu.MemorySpace.VMEM((3, 128, HH), jnp.float32),
            pltpu.SemaphoreType.DMA,
            pltpu.SemaphoreType.DMA,
            pltpu.SemaphoreType.DMA,
            pltpu.SemaphoreType.DMA,
            pltpu.SemaphoreType.DMA,
            pltpu.SemaphoreType.DMA,
            pltpu.SemaphoreType.DMA,
        ],
        compiler_params=_SC_PARAMS,
    )
    return kfn(srcp, dstp, l2d, m1, h01, acc0, den01)


@jax.jit
def kernel(x, edge_index, edge_attr, batch, W, att_src, att_dst, W_edge,
           att_edge, bias_conv, att_lin_W, att_lin_b, out_W, out_b):
    ae, csum = _edge_attr_proj(edge_attr, W_edge, att_edge)
    c = csum / jnp.float32(E)

    h0, h1, asrc, adst, sl = _node_proj(x, W, att_src, att_dst, c)

    src = edge_index[0]
    dst = edge_index[1]
    pad = EPAD - E
    srcp = jnp.pad(src, (0, pad)).reshape(ER, 128)
    dstp = jnp.pad(dst, (0, pad)).reshape(ER, 128)
    aep = jnp.pad(ae[:, 0], (0, pad), constant_values=PAD_AE).reshape(ER, 128)

    l2d, mpart = _run_pass1(srcp, dstp, aep,
                            asrc.reshape(N), adst.reshape(N))

    m, den0, a00, a01 = _combine_max(jnp.transpose(mpart), sl, h0, h1)

    h01 = jnp.concatenate([h0, h1], axis=0)
    acc0 = jnp.concatenate([a00, a01], axis=0)

    num, den = _run_pass2(srcp, dstp, l2d, m.reshape(N), h01, acc0,
                          den0.reshape(N))

    out, gate, mg = _node_out(num[:N], num[N:], den.reshape(N, 1),
                              bias_conv.reshape(1, H),
                              att_lin_W.reshape(1, H),
                              att_lin_b.reshape(1, 1),
                              batch.reshape(N, 1))

    _, _, y = _pool(out, gate, batch.reshape(N, 1), mg,
                    out_W, out_b.reshape(1, 1))
    return y
